# baseline (speedup 1.0000x reference)
import jax
import jax.numpy as jnp
from jax import lax
from jax.experimental import pallas as pl
from jax.experimental.pallas import tpu as pltpu

_B, _L, _DY, _DX, _DM = 8, 512, 32, 6, 128
_K = 6
_H = 16

_S0 = 0.99999986216691
_S1 = -0.16666607728014005
_S2 = 0.008332732437814282
_S3 = -0.0001981669232761085
_S4 = 2.708326132222227e-06
_S5 = -2.069597015432612e-08
_INV_2PI = 0.15915494309189535
_2PI_HI = 6.28125
_2PI_LO = 1.9353071795864769e-03


def _fast_sin(a):
    k = jnp.round(a * _INV_2PI)
    r = a - k * _2PI_HI - k * _2PI_LO
    r2 = r * r
    return r * (_S0 + r2 * (_S1 + r2 * (_S2 + r2 * (
        _S3 + r2 * (_S4 + r2 * _S5)))))


def _tc_body(x_ref, ys_ref, t2vw_ref, t2vb_ref, local_ref, vtw_ref, vtb_ref,
             space_ref, given_ref, out1_ref, out2_ref, out3_ref, base_ref):
    h = pl.program_id(0)
    b = pl.program_id(1)

    @pl.when(h == 0)
    def _compute_base():
        xs = x_ref[0]
        xs = jnp.where(jnp.isnan(xs), 0.0, xs)
        aff = xs * t2vw_ref[...] + t2vb_ref[...]
        col = lax.broadcasted_iota(jnp.int32, aff.shape, 1)
        te = jnp.where(col % _K == 0, aff, _fast_sin(aff))
        base_ref[pl.ds(b * _L, _L), :] = (
            local_ref[...] + vtb_ref[...] + given_ref[1:2, :]
            + jnp.dot(te, vtw_ref[0:_DX * _K, :],
                      preferred_element_type=jnp.float32))

    w_y = vtw_ref[_DX * _K:_DX * _K + 1, :]
    delta = given_ref[0:1, :] - given_ref[1:2, :]
    wd = jnp.concatenate([w_y, delta], axis=0)
    ys = ys_ref[0, 0]
    base = base_ref[pl.ds(b * _L, _L), :]
    for j in range(_H):
        yc = ys[:, j:j + 1]
        nan = jnp.isnan(yc)
        a = jnp.concatenate(
            [jnp.where(nan, 0.0, yc), jnp.where(nan, 1.0, 0.0)], axis=1)
        out1_ref[0, j * _L:(j + 1) * _L, :] = base + jnp.dot(
            a, wd, preferred_element_type=jnp.float32)
        out2_ref[0, j * _L:(j + 1) * _L, :] = jnp.broadcast_to(
            space_ref[j:j + 1, :], (_L, _DM))
        out3_ref[0, 0:1, j * _L:(j + 1) * _L] = jnp.full(
            (1, _L), h * _H + j, dtype=jnp.int32)


def kernel(x, y, t2v_w, t2v_b, local_table, vt_w, vt_b, space_table,
           given_table):
    batch, length, dy = y.shape
    x36 = jnp.repeat(x.reshape(batch, length, _DX), _K, axis=-1)
    ysplit = jnp.stack([y[:, :, :_H], y[:, :, _H:]])
    wflat = t2v_w.reshape(1, _DX * _K)
    bflat = t2v_b.reshape(1, _DX * _K)
    vtb2 = vt_b.reshape(1, _DM)

    out1, out2, out3 = pl.pallas_call(
        _tc_body,
        grid=(dy // _H, batch),
        in_specs=[
            pl.BlockSpec((1, length, _DX * _K), lambda h, b: (b, 0, 0)),
            pl.BlockSpec((1, 1, length, _H), lambda h, b: (h, b, 0, 0)),
            pl.BlockSpec((1, _DX * _K), lambda h, b: (0, 0)),
            pl.BlockSpec((1, _DX * _K), lambda h, b: (0, 0)),
            pl.BlockSpec((length, _DM), lambda h, b: (0, 0)),
            pl.BlockSpec((_DX * _K + 1, _DM), lambda h, b: (0, 0)),
            pl.BlockSpec((1, _DM), lambda h, b: (0, 0)),
            pl.BlockSpec((_H, _DM), lambda h, b: (h, 0)),
            pl.BlockSpec((2, _DM), lambda h, b: (0, 0)),
        ],
        out_specs=[
            pl.BlockSpec((1, _H * length, _DM), lambda h, b: (b, h, 0)),
            pl.BlockSpec((1, _H * length, _DM), lambda h, b: (b, h, 0)),
            pl.BlockSpec((1, 1, _H * length), lambda h, b: (b, 0, h)),
        ],
        out_shape=[
            jax.ShapeDtypeStruct((batch, dy * length, _DM), jnp.float32),
            jax.ShapeDtypeStruct((batch, dy * length, _DM), jnp.float32),
            jax.ShapeDtypeStruct((batch, 1, dy * length), jnp.int32),
        ],
        scratch_shapes=[pltpu.VMEM((batch * length, _DM), jnp.float32)],
        compiler_params=pltpu.CompilerParams(
            dimension_semantics=("arbitrary", "arbitrary")),
    )(x36, ysplit, wflat, bflat, local_table[:length], vt_w, vtb2,
      space_table, given_table)

    return (out1, out2, out3.reshape(batch, dy * length))


# speedup vs baseline: 1.0029x; 1.0029x over previous
import jax
import jax.numpy as jnp
from jax import lax
from jax.experimental import pallas as pl
from jax.experimental.pallas import tpu as pltpu

_B, _L, _DY, _DX, _DM = 8, 512, 32, 6, 128
_K = 6

_S0 = 0.99999986216691
_S1 = -0.16666607728014005
_S2 = 0.008332732437814282
_S3 = -0.0001981669232761085
_S4 = 2.708326132222227e-06
_S5 = -2.069597015432612e-08
_INV_2PI = 0.15915494309189535
_2PI_HI = 6.28125
_2PI_LO = 1.9353071795864769e-03


def _fast_sin(a):
    k = jnp.round(a * _INV_2PI)
    r = a - k * _2PI_HI - k * _2PI_LO
    r2 = r * r
    return r * (_S0 + r2 * (_S1 + r2 * (_S2 + r2 * (
        _S3 + r2 * (_S4 + r2 * _S5)))))


def _tc_body(x_ref, y_ref, t2vw_ref, t2vb_ref, local_ref, vtw_ref, vtb_ref,
             space_ref, given_ref, out1_ref, out2_ref, out3_ref, base_ref):
    xs6 = x_ref[0]
    xs6 = jnp.where(jnp.isnan(xs6), 0.0, xs6)
    xs = jnp.concatenate(
        [jnp.broadcast_to(xs6[:, i:i + 1], (_L, _K)) for i in range(_DX)],
        axis=1)
    aff = xs * t2vw_ref[...] + t2vb_ref[...]
    col = lax.broadcasted_iota(jnp.int32, aff.shape, 1)
    te = jnp.where(col % _K == 0, aff, _fast_sin(aff))
    base_ref[...] = (local_ref[...] + vtb_ref[...] + given_ref[1:2, :]
                     + jnp.dot(te, vtw_ref[0:_DX * _K, :],
                               preferred_element_type=jnp.float32))

    w_y = vtw_ref[_DX * _K:_DX * _K + 1, :]
    delta = given_ref[0:1, :] - given_ref[1:2, :]
    wd = jnp.concatenate([w_y, delta], axis=0)
    ys = y_ref[0]
    base = base_ref[...]
    for j in range(_DY):
        yc = ys[:, j:j + 1]
        nan = jnp.isnan(yc)
        a = jnp.concatenate(
            [jnp.where(nan, 0.0, yc), jnp.where(nan, 1.0, 0.0)], axis=1)
        out1_ref[0, j * _L:(j + 1) * _L, :] = base + jnp.dot(
            a, wd, preferred_element_type=jnp.float32)
        out2_ref[0, j * _L:(j + 1) * _L, :] = jnp.broadcast_to(
            space_ref[j:j + 1, :], (_L, _DM))
        out3_ref[0, 0:1, j * _L:(j + 1) * _L] = jnp.full(
            (1, _L), j, dtype=jnp.int32)


def kernel(x, y, t2v_w, t2v_b, local_table, vt_w, vt_b, space_table,
           given_table):
    batch, length, dy = y.shape
    x3 = x.reshape(batch, length, _DX)
    wflat = t2v_w.reshape(1, _DX * _K)
    bflat = t2v_b.reshape(1, _DX * _K)
    vtb2 = vt_b.reshape(1, _DM)

    out1, out2, out3 = pl.pallas_call(
        _tc_body,
        grid=(batch,),
        in_specs=[
            pl.BlockSpec((1, length, _DX), lambda b: (b, 0, 0)),
            pl.BlockSpec((1, length, dy), lambda b: (b, 0, 0)),
            pl.BlockSpec((1, _DX * _K), lambda b: (0, 0)),
            pl.BlockSpec((1, _DX * _K), lambda b: (0, 0)),
            pl.BlockSpec((length, _DM), lambda b: (0, 0)),
            pl.BlockSpec((_DX * _K + 1, _DM), lambda b: (0, 0)),
            pl.BlockSpec((1, _DM), lambda b: (0, 0)),
            pl.BlockSpec((_DY, _DM), lambda b: (0, 0)),
            pl.BlockSpec((2, _DM), lambda b: (0, 0)),
        ],
        out_specs=[
            pl.BlockSpec((1, dy * length, _DM), lambda b: (b, 0, 0)),
            pl.BlockSpec((1, dy * length, _DM), lambda b: (b, 0, 0)),
            pl.BlockSpec((1, 1, dy * length), lambda b: (b, 0, 0)),
        ],
        out_shape=[
            jax.ShapeDtypeStruct((batch, dy * length, _DM), jnp.float32),
            jax.ShapeDtypeStruct((batch, dy * length, _DM), jnp.float32),
            jax.ShapeDtypeStruct((batch, 1, dy * length), jnp.int32),
        ],
        scratch_shapes=[pltpu.VMEM((length, _DM), jnp.float32)],
        compiler_params=pltpu.CompilerParams(
            dimension_semantics=("arbitrary",)),
    )(x3, y, wflat, bflat, local_table[:length], vt_w, vtb2, space_table,
      given_table)

    return (out1, out2, out3.reshape(batch, dy * length))


# speedup vs baseline: 1.0216x; 1.0187x over previous
import jax
import jax.numpy as jnp
from jax import lax
from jax.experimental import pallas as pl
from jax.experimental.pallas import tpu as pltpu

_B, _L, _DY, _DX, _DM = 8, 512, 32, 6, 128
_K = 6

_S0 = 0.99999986216691
_S1 = -0.16666607728014005
_S2 = 0.008332732437814282
_S3 = -0.0001981669232761085
_S4 = 2.708326132222227e-06
_S5 = -2.069597015432612e-08
_INV_2PI = 0.15915494309189535
_2PI_HI = 6.28125
_2PI_LO = 1.9353071795864769e-03


def _fast_sin(a):
    k = jnp.round(a * _INV_2PI)
    r = a - k * _2PI_HI - k * _2PI_LO
    r2 = r * r
    return r * (_S0 + r2 * (_S1 + r2 * (_S2 + r2 * (
        _S3 + r2 * (_S4 + r2 * _S5)))))


def _tc_body(x_ref, y_ref, t2vw_ref, t2vb_ref, local_ref, vtw_ref, vtb_ref,
             space_ref, given_ref, out1_ref, out2_ref, out3_ref, base_ref):
    xs = x_ref[0]
    xs = jnp.where(jnp.isnan(xs), 0.0, xs)
    aff = xs * t2vw_ref[...] + t2vb_ref[...]
    col = lax.broadcasted_iota(jnp.int32, aff.shape, 1)
    te = jnp.where(col % _K == 0, aff, _fast_sin(aff))
    base_ref[...] = (local_ref[...] + vtb_ref[...] + given_ref[1:2, :]
                     + jnp.dot(te, vtw_ref[0:_DX * _K, :],
                               preferred_element_type=jnp.float32))

    w_y = vtw_ref[_DX * _K:_DX * _K + 1, :]
    delta = given_ref[0:1, :] - given_ref[1:2, :]
    wd = jnp.concatenate([w_y, delta], axis=0)
    ys = y_ref[0]
    base = base_ref[...]
    for j in range(_DY):
        yc = ys[:, j:j + 1]
        nan = jnp.isnan(yc)
        a = jnp.concatenate(
            [jnp.where(nan, 0.0, yc), jnp.where(nan, 1.0, 0.0)], axis=1)
        out1_ref[0, j * _L:(j + 1) * _L, :] = base + jnp.dot(
            a, wd, preferred_element_type=jnp.float32)
        out2_ref[0, j * _L:(j + 1) * _L, :] = jnp.broadcast_to(
            space_ref[j:j + 1, :], (_L, _DM))
        out3_ref[0, 0:1, j * _L:(j + 1) * _L] = jnp.full(
            (1, _L), j, dtype=jnp.int32)


def kernel(x, y, t2v_w, t2v_b, local_table, vt_w, vt_b, space_table,
           given_table):
    batch, length, dy = y.shape
    x36 = jnp.repeat(x.reshape(batch, length, _DX), _K, axis=-1)
    wflat = t2v_w.reshape(1, _DX * _K)
    bflat = t2v_b.reshape(1, _DX * _K)
    vtb2 = vt_b.reshape(1, _DM)

    out1, out2, out3 = pl.pallas_call(
        _tc_body,
        grid=(batch,),
        in_specs=[
            pl.BlockSpec((1, length, _DX * _K), lambda b: (b, 0, 0)),
            pl.BlockSpec((1, length, dy), lambda b: (b, 0, 0)),
            pl.BlockSpec((1, _DX * _K), lambda b: (0, 0)),
            pl.BlockSpec((1, _DX * _K), lambda b: (0, 0)),
            pl.BlockSpec((length, _DM), lambda b: (0, 0)),
            pl.BlockSpec((_DX * _K + 1, _DM), lambda b: (0, 0)),
            pl.BlockSpec((1, _DM), lambda b: (0, 0)),
            pl.BlockSpec((_DY, _DM), lambda b: (0, 0)),
            pl.BlockSpec((2, _DM), lambda b: (0, 0)),
        ],
        out_specs=[
            pl.BlockSpec((1, dy * length, _DM), lambda b: (b, 0, 0)),
            pl.BlockSpec((1, dy * length, _DM), lambda b: (b, 0, 0)),
            pl.BlockSpec((1, 1, dy * length), lambda b: (b, 0, 0)),
        ],
        out_shape=[
            jax.ShapeDtypeStruct((batch, dy * length, _DM), jnp.float32),
            jax.ShapeDtypeStruct((batch, dy * length, _DM), jnp.float32),
            jax.ShapeDtypeStruct((batch, 1, dy * length), jnp.int32),
        ],
        scratch_shapes=[pltpu.VMEM((length, _DM), jnp.float32)],
        compiler_params=pltpu.CompilerParams(
            dimension_semantics=("arbitrary",)),
    )(x36, y, wflat, bflat, local_table[:length], vt_w, vtb2, space_table,
      given_table)

    return (out1, out2, out3.reshape(batch, dy * length))
